# Initial kernel scaffold; baseline (speedup 1.0000x reference)
import jax
import jax.numpy as jnp
from jax.experimental import pallas as pl


def kernel(src, dst, embedding):
    raise NotImplementedError("write your pallas kernel here")


# speedup vs baseline: 1.3969x; 1.3969x over previous
import functools
import jax
import jax.numpy as jnp
from jax import lax
from jax.experimental import pallas as pl
from jax.experimental.pallas import tpu as pltpu
from jax.experimental.pallas import tpu_sc as plsc

_B = 1_000_000
_D = 64
_NC = 2
_NS = 16
_NW = _NC * _NS
_C = 320
_G = 80
_NG = _C // _G
_NCHUNK = _B // _C
_TMAX = -(-_NCHUNK // _NW)
_NBLK = _C // 16


def _body(src_hbm, dst_hbm, table_hbm, out_hbm,
          sidx_v, didx_v, srows_v, drows_v, out_v, sem):
    wid = lax.axis_index("s") * _NC + lax.axis_index("c")
    lane = lax.iota(jnp.int32, 16)

    def chunk_body(t, _):
        c = wid + t * _NW

        @pl.when(c < _NCHUNK)
        def _():
            base = c * _C
            pltpu.sync_copy(src_hbm.at[pl.ds(base, _C)], sidx_v)
            pltpu.sync_copy(dst_hbm.at[pl.ds(base, _C)], didx_v)
            s2d = srows_v
            d2d = drows_v
            for j in range(_NG):
                pltpu.async_copy(
                    table_hbm.at[sidx_v.at[pl.ds(j * _G, _G)]],
                    s2d.at[pl.ds(j * _G, _G)], sem)
                pltpu.async_copy(
                    table_hbm.at[didx_v.at[pl.ds(j * _G, _G)]],
                    d2d.at[pl.ds(j * _G, _G)], sem)
            for j in range(_NG):
                pltpu.make_async_copy(
                    table_hbm.at[sidx_v.at[pl.ds(j * _G, _G)]],
                    s2d.at[pl.ds(j * _G, _G)], sem).wait()
                pltpu.make_async_copy(
                    table_hbm.at[didx_v.at[pl.ds(j * _G, _G)]],
                    d2d.at[pl.ds(j * _G, _G)], sem).wait()

            def samp_body(i, _):
                acc = jnp.zeros((16,), jnp.float32)
                for k in range(_D // 16):
                    a = srows_v[i, pl.ds(k * 16, 16)]
                    bb = drows_v[i, pl.ds(k * 16, 16)]
                    acc = acc + jnp.abs(a - bb)
                tot = jnp.sum(acc)
                plsc.store_scatter(out_v, [jnp.full((16,), i, jnp.int32)],
                                   jnp.full((16,), tot, jnp.float32),
                                   mask=lane == 0)
                return 0

            lax.fori_loop(0, _C, samp_body, 0)
            pltpu.sync_copy(out_v, out_hbm.at[pl.ds(base, _C)])

        return 0

    lax.fori_loop(0, _TMAX, chunk_body, 0)


@jax.jit
def _emb_l1(src, dst, table):
    mesh = plsc.VectorSubcoreMesh(core_axis_name="c", subcore_axis_name="s")
    f = pl.kernel(
        _body,
        out_type=jax.ShapeDtypeStruct((_B,), jnp.float32),
        mesh=mesh,
        scratch_types=[
            pltpu.VMEM((_C,), jnp.int32),
            pltpu.VMEM((_C,), jnp.int32),
            pltpu.VMEM((_C, _D), jnp.float32),
            pltpu.VMEM((_C, _D), jnp.float32),
            pltpu.VMEM((_C,), jnp.float32),
            pltpu.SemaphoreType.DMA,
        ],
        compiler_params=pltpu.CompilerParams(
            needs_layout_passes=False, use_tc_tiling_on_sc=False),
        name="emb_l1_sc",
    )
    return f(src, dst, table)


def kernel(src, dst, embedding):
    return _emb_l1(src, dst, embedding)
